# Initial kernel scaffold; baseline (speedup 1.0000x reference)
import jax
import jax.numpy as jnp
from jax.experimental import pallas as pl


def kernel(x, edge_index, edge_attr, masked_atom_mask, params):
    raise NotImplementedError("write your pallas kernel here")


# speedup vs baseline: 6.6429x; 6.6429x over previous
import functools

import numpy as np

import jax
import jax.numpy as jnp
from jax import lax
from jax.experimental import pallas as pl
from jax.experimental.pallas import tpu as pltpu
from jax.experimental.pallas import tpu_sc as plsc

_N = 10000
_E = 160000
_EMB = 500
_EMBP = 512
_HIDP = 1024
_OUT = 119
_CH = 128
_NCH = 4
_TILES = 16
_BLK = 128
_EPT = _E // _TILES
_NB = (_EPT + _BLK - 1) // _BLK
_EPTP = _NB * _BLK
_NPAD = 10112
_DUMMY = _N
_STRIPE = _NPAD // _TILES
_OSTRIPE = 624
_OTAIL = _N - _TILES * _OSTRIPE
_RB = 400
_GRID = _N // _RB

@functools.lru_cache(maxsize=None)
def _build_spmv(nch):
    mesh = plsc.VectorSubcoreMesh(core_axis_name="c", subcore_axis_name="s")
    return functools.partial(
        pl.kernel,
        out_type=[jax.ShapeDtypeStruct((_N, _CH), jnp.float32)] * nch,
        mesh=mesh,
        scratch_types=[
            pltpu.VMEM((_NB, _BLK), jnp.int32),
            pltpu.VMEM((_NB, _BLK), jnp.int32),
            pltpu.VMEM((_BLK, _CH), jnp.float32),
            pltpu.VMEM_SHARED((_NPAD, _CH), jnp.float32),
            pltpu.SemaphoreType.DMA,
        ],
    )(functools.partial(_spmv_body, nch))


def _sc_spmv(h0, h1, h2, h3, srcp, dstp):
    return _build_spmv(_NCH)(h0, h1, h2, h3, srcp, dstp)


def _sc_cnt(tab, combop, dstp):
    return _build_spmv(1)(tab, combop, dstp)[0]


def _spmv_body(nch, *refs):
    hrefs = refs[0:nch]
    srcp, dstp = refs[nch:nch + 2]
    orefs = refs[nch + 2:2 * nch + 2]
    src_v, dst_v, gbuf, acc, sem = refs[2 * nch + 2:]
    cid = lax.axis_index("c")
    sid = lax.axis_index("s")
    pltpu.sync_copy(srcp.at[sid], src_v)
    pltpu.sync_copy(dstp.at[sid], dst_v)

    zero16 = jnp.zeros((16,), jnp.float32)

    for ch in range(nch):
        @pl.when(cid == ch // 2)
        def _(ch=ch):
            def _zrow(r, carry):
                for j in range(_CH // 16):
                    gbuf[r, pl.ds(j * 16, 16)] = zero16
                return carry

            lax.fori_loop(0, _BLK, _zrow, 0)
            base = sid * _STRIPE
            nfull = _STRIPE // _BLK
            for z in range(nfull):
                pltpu.sync_copy(gbuf, acc.at[pl.ds(base + z * _BLK, _BLK)])
            rem = _STRIPE - nfull * _BLK
            if rem:
                pltpu.sync_copy(gbuf.at[pl.ds(0, rem)],
                                acc.at[pl.ds(base + nfull * _BLK, rem)])
            plsc.subcore_barrier()

            tab = hrefs[ch]

            def _blk(j, carry):
                pltpu.async_copy(tab.at[src_v.at[j]], gbuf, sem).wait()
                pltpu.sync_copy(gbuf, acc.at[dst_v.at[j]], add=True)
                return carry

            lax.fori_loop(0, _NB, _blk, 0)
            plsc.subcore_barrier()
            pltpu.sync_copy(acc.at[pl.ds(sid * _OSTRIPE, _OSTRIPE)],
                            orefs[ch].at[pl.ds(sid * _OSTRIPE, _OSTRIPE)])

            @pl.when(sid == _TILES - 1)
            def _tail(ch=ch):
                pltpu.sync_copy(
                    acc.at[pl.ds(_TILES * _OSTRIPE, _OTAIL)],
                    orefs[ch].at[pl.ds(_TILES * _OSTRIPE, _OTAIL)])
            plsc.subcore_barrier()


def _full(shape):
    return pl.BlockSpec(shape, lambda i: (0,) * len(shape))


def _rows(shape):
    return pl.BlockSpec(shape, lambda i: (i,) + (0,) * (len(shape) - 1))


def _tc_embed(x0, x1, e1, e2):
    def body(x0r, x1r, e1r, e2r, o0, o1, o2, o3):
        oh1 = (x0r[...] == lax.broadcasted_iota(jnp.int32, (1, 128), 1))
        oh2 = (x1r[...] == lax.broadcasted_iota(jnp.int32, (1, 8), 1))
        h = jnp.dot(oh1.astype(jnp.float32), e1r[...],
                    preferred_element_type=jnp.float32)
        h = h + jnp.dot(oh2.astype(jnp.float32), e2r[...],
                        preferred_element_type=jnp.float32)
        for k, o in enumerate((o0, o1, o2, o3)):
            o[...] = h[:, k * _CH:(k + 1) * _CH]

    return pl.pallas_call(
        body,
        grid=(_GRID,),
        in_specs=[_rows((_RB, 1)), _rows((_RB, 1)),
                  _full((128, _EMBP)), _full((8, _EMBP))],
        out_specs=[_rows((_RB, _CH))] * _NCH,
        out_shape=[jax.ShapeDtypeStruct((_N, _CH), jnp.float32)] * _NCH,
    )(x0, x1, e1, e2)


def _tc_dense(z, h, cnt, ee, sl, w1, b1, w2, b2, bns, bnb, relu_out, nout):
    def body(*refs):
        zr = refs[0:4]
        hr = refs[4:8]
        cr, eer, slr, w1r, b1r, w2r, b2r = refs[8:15]
        pos = 15
        if bns is not None:
            bnsr, bnbr = refs[15:17]
            pos = 17
        outs = refs[pos:]
        agg = jnp.concatenate([a[...] + b[...] for a, b in zip(zr, hr)],
                              axis=1)
        agg = agg + slr[...] + jnp.dot(cr[...], eer[...],
                                       preferred_element_type=jnp.float32)
        hid = jnp.maximum(jnp.dot(agg, w1r[...],
                                  preferred_element_type=jnp.float32)
                          + b1r[...], 0.0)
        o = jnp.dot(hid, w2r[...], preferred_element_type=jnp.float32) \
            + b2r[...]
        if bns is not None:
            o = o * bnsr[...] + bnbr[...]
        if relu_out:
            o = jnp.maximum(o, 0.0)
        for k, oref in enumerate(outs):
            oref[...] = o[:, k * _CH:(k + 1) * _CH]

    ncols = nout * _CH
    in_specs = ([_rows((_RB, _CH))] * 8
                + [_rows((_RB, _CH)), _full((_CH, _EMBP)), _full((1, _EMBP)),
                   _full((_EMBP, _HIDP)), _full((1, _HIDP)),
                   _full((_HIDP, ncols)), _full((1, ncols))])
    args = list(z) + list(h) + [cnt, ee, sl, w1, b1, w2, b2]
    if bns is not None:
        in_specs += [_full((1, ncols)), _full((1, ncols))]
        args += [bns, bnb]
    return pl.pallas_call(
        body,
        grid=(_GRID,),
        in_specs=in_specs,
        out_specs=[_rows((_RB, _CH))] * nout,
        out_shape=[jax.ShapeDtypeStruct((_N, _CH), jnp.float32)] * nout,
    )(*args)


def _tc_decpre(h, mask, e2d, alpha):
    def body(h0r, h1r, h2r, h3r, mr, er, ar, o0, o1, o2, o3):
        hcat = jnp.concatenate([r[...] for r in (h0r, h1r, h2r, h3r)], axis=1)
        a = ar[0, 0]
        p = jnp.where(hcat >= 0.0, hcat, a * hcat)
        d = jnp.dot(p, er[...], preferred_element_type=jnp.float32)
        d = jnp.where(mr[...] > 0, 0.0, d)
        for k, o in enumerate((o0, o1, o2, o3)):
            o[...] = d[:, k * _CH:(k + 1) * _CH]

    return pl.pallas_call(
        body,
        grid=(_GRID,),
        in_specs=[_rows((_RB, _CH))] * 4
        + [_rows((_RB, 1)), _full((_EMBP, _EMBP)), _full((1, 1))],
        out_specs=[_rows((_RB, _CH))] * _NCH,
        out_shape=[jax.ShapeDtypeStruct((_N, _CH), jnp.float32)] * _NCH,
    )(*h, mask, e2d, alpha)


def _pad2(a, r, c):
    return jnp.pad(a.astype(jnp.float32), ((0, r - a.shape[0]),
                                           (0, c - a.shape[1])))


def _edge_slabs(v, fill):
    v = v.astype(jnp.int32).reshape(_TILES, _EPT)
    v = jnp.pad(v, ((0, 0), (0, _EPTP - _EPT)), constant_values=fill)
    return v.reshape(_TILES, _NB, _BLK)


def _ee_matrix(ee1, ee2):
    m = jnp.zeros((_CH, _EMBP), jnp.float32)
    m = m.at[0:6, :_EMB].set(ee1.astype(jnp.float32))
    m = m.at[8:11, :_EMB].set(ee2.astype(jnp.float32))
    return m


def kernel(x, edge_index, edge_attr, masked_atom_mask, params):
    x0 = x[:, 0].astype(jnp.int32).reshape(_N, 1)
    x1 = x[:, 1].astype(jnp.int32).reshape(_N, 1)
    srcp = _edge_slabs(edge_index[0], 0)
    dstp = _edge_slabs(edge_index[1], _DUMMY)
    combop = _edge_slabs(edge_attr[:, 0] * 3 + edge_attr[:, 1], 0)
    maskcol = masked_atom_mask.astype(jnp.int32).reshape(_N, 1)

    p = params
    e1 = _pad2(p['atom_emb1'], 128, _EMBP)
    e2 = _pad2(p['atom_emb2'], 8, _EMBP)
    e2d = _pad2(p['enc_to_dec'], _EMBP, _EMBP)
    alpha = p['prelu_alpha'].reshape(1, 1).astype(jnp.float32)

    t = np.zeros((24, _CH), np.float32)
    for k in range(9):
        t[k, k // 3] = 1.0
        t[k, 8 + k % 3] = 1.0
    cnt = _sc_cnt(jnp.asarray(t), combop, dstp)

    h = _tc_embed(x0, x1, e1, e2)
    for l, lp in enumerate(p['layers']):
        z = _sc_spmv(*h, srcp, dstp)
        ee = _ee_matrix(lp['ee1'], lp['ee2'])
        sl = jnp.pad((lp['ee1'][4] + lp['ee2'][0]).astype(jnp.float32),
                     (0, _EMBP - _EMB)).reshape(1, _EMBP)
        w1 = _pad2(lp['W1'], _EMBP, _HIDP)
        b1 = jnp.pad(lp['b1'].astype(jnp.float32),
                     (0, _HIDP - 2 * _EMB)).reshape(1, _HIDP)
        w2 = _pad2(lp['W2'], _HIDP, _EMBP)
        b2 = jnp.pad(lp['b2'].astype(jnp.float32),
                     (0, _EMBP - _EMB)).reshape(1, _EMBP)
        bns = jnp.pad(lp['bn_scale'].astype(jnp.float32),
                      (0, _EMBP - _EMB)).reshape(1, _EMBP)
        bnb = jnp.pad(lp['bn_bias'].astype(jnp.float32),
                      (0, _EMBP - _EMB)).reshape(1, _EMBP)
        h = _tc_dense(z, h, cnt, ee, sl, w1, b1, w2, b2, bns, bnb,
                      relu_out=(l != 4), nout=_NCH)

    d = _tc_decpre(h, maskcol, e2d, alpha)
    z = _sc_spmv(*d, srcp, dstp)
    dp = p['dec']
    ee = _ee_matrix(dp['ee1'], dp['ee2'])
    sl = jnp.pad((dp['ee1'][4] + dp['ee2'][0]).astype(jnp.float32),
                 (0, _EMBP - _EMB)).reshape(1, _EMBP)
    w1 = _pad2(dp['W1'], _EMBP, _HIDP)
    b1 = jnp.pad(dp['b1'].astype(jnp.float32),
                 (0, _HIDP - 2 * _EMB)).reshape(1, _HIDP)
    w2 = _pad2(dp['W2'], _HIDP, _CH)
    b2 = jnp.pad(dp['b2'].astype(jnp.float32), (0, _CH - _OUT)).reshape(1, _CH)
    out = _tc_dense(z, d, cnt, ee, sl, w1, b1, w2, b2, None, None,
                    relu_out=False, nout=1)
    return out[0][:, :_OUT]


# speedup vs baseline: 8.3287x; 1.2538x over previous
import functools

import numpy as np

import jax
import jax.numpy as jnp
from jax import lax
from jax.experimental import pallas as pl
from jax.experimental.pallas import tpu as pltpu
from jax.experimental.pallas import tpu_sc as plsc

_N = 10000
_E = 160000
_EMB = 500
_EMBP = 512
_HIDP = 1024
_OUT = 119
_CH = 128
_NCH = 4
_TILES = 16
_BLK = 128
_EPT = _E // _TILES
_NB = (_EPT + _BLK - 1) // _BLK
_EPTP = _NB * _BLK
_NPAD = 10112
_DUMMY = _N
_STRIPE = _NPAD // _TILES
_OSTRIPE = 624
_OTAIL = _N - _TILES * _OSTRIPE
_RB = 400
_GRID = _N // _RB

@functools.lru_cache(maxsize=None)
def _build_spmv(nch):
    mesh = plsc.VectorSubcoreMesh(core_axis_name="c", subcore_axis_name="s")
    return functools.partial(
        pl.kernel,
        out_type=[jax.ShapeDtypeStruct((_N, _CH), jnp.float32)] * nch,
        mesh=mesh,
        scratch_types=[
            pltpu.VMEM((_NB, _BLK), jnp.int32),
            pltpu.VMEM((4, _BLK), jnp.int32),
            pltpu.VMEM((_BLK, _CH), jnp.float32),
            pltpu.VMEM((_BLK, _CH), jnp.float32),
            pltpu.VMEM_SHARED((_NPAD, _CH), jnp.float32),
            pltpu.SemaphoreType.DMA,
            pltpu.SemaphoreType.DMA,
        ],
    )(functools.partial(_spmv_body, nch))


def _sc_spmv(h0, h1, h2, h3, pkp):
    return _build_spmv(_NCH)(h0, h1, h2, h3, pkp)


def _sc_cnt(tab, pkp):
    return _build_spmv(1)(tab, pkp)[0]


def _spmv_body(nch, *refs):
    hrefs = refs[0:nch]
    pkp = refs[nch]
    orefs = refs[nch + 1:2 * nch + 1]
    pk_v, ibuf, gb0, gb1, acc, sem0, sem1 = refs[2 * nch + 1:]
    cid = lax.axis_index("c")
    sid = lax.axis_index("s")
    pltpu.sync_copy(pkp.at[sid], pk_v)

    zero16 = jnp.zeros((16,), jnp.float32)
    m16 = jnp.full((16,), 0xFFFF, jnp.int32)

    def _unpack(j, slot):
        for g in range(_BLK // 16):
            v = pk_v[j, pl.ds(g * 16, 16)]
            ibuf[2 * slot, pl.ds(g * 16, 16)] = jnp.bitwise_and(v, m16)
            ibuf[2 * slot + 1, pl.ds(g * 16, 16)] = \
                lax.shift_right_logical(v, 16)

    for ch in range(nch):
        @pl.when(cid == ch // 2)
        def _(ch=ch):
            def _zrow(r, carry):
                for j in range(_CH // 16):
                    gb0[r, pl.ds(j * 16, 16)] = zero16
                return carry

            lax.fori_loop(0, _BLK, _zrow, 0)
            base = sid * _STRIPE
            nfull = _STRIPE // _BLK
            for z in range(nfull):
                pltpu.sync_copy(gb0, acc.at[pl.ds(base + z * _BLK, _BLK)])
            rem = _STRIPE - nfull * _BLK
            if rem:
                pltpu.sync_copy(gb0.at[pl.ds(0, rem)],
                                acc.at[pl.ds(base + nfull * _BLK, rem)])
            plsc.subcore_barrier()

            tab = hrefs[ch]

            def _wait(buf, sem):
                pltpu.make_async_copy(tab.at[ibuf.at[0]], buf, sem).wait()

            _unpack(0, 0)
            pltpu.async_copy(tab.at[ibuf.at[0]], gb0, sem0)

            def _pair(p, carry):
                j0 = 2 * p
                _unpack(j0 + 1, 1)
                pltpu.async_copy(tab.at[ibuf.at[2]], gb1, sem1)
                _wait(gb0, sem0)
                pltpu.sync_copy(gb0, acc.at[ibuf.at[1]], add=True)
                _unpack(j0 + 2, 0)
                pltpu.async_copy(tab.at[ibuf.at[0]], gb0, sem0)
                _wait(gb1, sem1)
                pltpu.sync_copy(gb1, acc.at[ibuf.at[3]], add=True)
                return carry

            lax.fori_loop(0, (_NB - 1) // 2, _pair, 0)
            _wait(gb0, sem0)
            pltpu.sync_copy(gb0, acc.at[ibuf.at[1]], add=True)
            plsc.subcore_barrier()
            pltpu.sync_copy(acc.at[pl.ds(sid * _OSTRIPE, _OSTRIPE)],
                            orefs[ch].at[pl.ds(sid * _OSTRIPE, _OSTRIPE)])

            @pl.when(sid == _TILES - 1)
            def _tail(ch=ch):
                pltpu.sync_copy(
                    acc.at[pl.ds(_TILES * _OSTRIPE, _OTAIL)],
                    orefs[ch].at[pl.ds(_TILES * _OSTRIPE, _OTAIL)])
            plsc.subcore_barrier()


def _full(shape):
    return pl.BlockSpec(shape, lambda i: (0,) * len(shape))


def _rows(shape):
    return pl.BlockSpec(shape, lambda i: (i,) + (0,) * (len(shape) - 1))


def _tc_embed(x0, x1, e1, e2):
    def body(x0r, x1r, e1r, e2r, o0, o1, o2, o3):
        oh1 = (x0r[...] == lax.broadcasted_iota(jnp.int32, (1, 128), 1))
        oh2 = (x1r[...] == lax.broadcasted_iota(jnp.int32, (1, 8), 1))
        h = jnp.dot(oh1.astype(jnp.float32), e1r[...],
                    preferred_element_type=jnp.float32)
        h = h + jnp.dot(oh2.astype(jnp.float32), e2r[...],
                        preferred_element_type=jnp.float32)
        for k, o in enumerate((o0, o1, o2, o3)):
            o[...] = h[:, k * _CH:(k + 1) * _CH]

    return pl.pallas_call(
        body,
        grid=(_GRID,),
        in_specs=[_rows((_RB, 1)), _rows((_RB, 1)),
                  _full((128, _EMBP)), _full((8, _EMBP))],
        out_specs=[_rows((_RB, _CH))] * _NCH,
        out_shape=[jax.ShapeDtypeStruct((_N, _CH), jnp.float32)] * _NCH,
    )(x0, x1, e1, e2)


def _tc_dense(z, h, cnt, ee, sl, w1, b1, w2, b2, bns, bnb, relu_out, nout):
    def body(*refs):
        zr = refs[0:4]
        hr = refs[4:8]
        cr, eer, slr, w1r, b1r, w2r, b2r = refs[8:15]
        pos = 15
        if bns is not None:
            bnsr, bnbr = refs[15:17]
            pos = 17
        outs = refs[pos:]
        agg = jnp.concatenate([a[...] + b[...] for a, b in zip(zr, hr)],
                              axis=1)
        agg = agg + slr[...] + jnp.dot(cr[...], eer[...],
                                       preferred_element_type=jnp.float32)
        hid = jnp.maximum(jnp.dot(agg, w1r[...],
                                  preferred_element_type=jnp.float32)
                          + b1r[...], 0.0)
        o = jnp.dot(hid, w2r[...], preferred_element_type=jnp.float32) \
            + b2r[...]
        if bns is not None:
            o = o * bnsr[...] + bnbr[...]
        if relu_out:
            o = jnp.maximum(o, 0.0)
        for k, oref in enumerate(outs):
            oref[...] = o[:, k * _CH:(k + 1) * _CH]

    ncols = nout * _CH
    in_specs = ([_rows((_RB, _CH))] * 8
                + [_rows((_RB, _CH)), _full((_CH, _EMBP)), _full((1, _EMBP)),
                   _full((_EMBP, _HIDP)), _full((1, _HIDP)),
                   _full((_HIDP, ncols)), _full((1, ncols))])
    args = list(z) + list(h) + [cnt, ee, sl, w1, b1, w2, b2]
    if bns is not None:
        in_specs += [_full((1, ncols)), _full((1, ncols))]
        args += [bns, bnb]
    return pl.pallas_call(
        body,
        grid=(_GRID,),
        in_specs=in_specs,
        out_specs=[_rows((_RB, _CH))] * nout,
        out_shape=[jax.ShapeDtypeStruct((_N, _CH), jnp.float32)] * nout,
    )(*args)


def _tc_decpre(h, mask, e2d, alpha):
    def body(h0r, h1r, h2r, h3r, mr, er, ar, o0, o1, o2, o3):
        hcat = jnp.concatenate([r[...] for r in (h0r, h1r, h2r, h3r)], axis=1)
        a = ar[0, 0]
        p = jnp.where(hcat >= 0.0, hcat, a * hcat)
        d = jnp.dot(p, er[...], preferred_element_type=jnp.float32)
        d = jnp.where(mr[...] > 0, 0.0, d)
        for k, o in enumerate((o0, o1, o2, o3)):
            o[...] = d[:, k * _CH:(k + 1) * _CH]

    return pl.pallas_call(
        body,
        grid=(_GRID,),
        in_specs=[_rows((_RB, _CH))] * 4
        + [_rows((_RB, 1)), _full((_EMBP, _EMBP)), _full((1, 1))],
        out_specs=[_rows((_RB, _CH))] * _NCH,
        out_shape=[jax.ShapeDtypeStruct((_N, _CH), jnp.float32)] * _NCH,
    )(*h, mask, e2d, alpha)


def _pad2(a, r, c):
    return jnp.pad(a.astype(jnp.float32), ((0, r - a.shape[0]),
                                           (0, c - a.shape[1])))


def _packed_slabs(srcv, dstv):
    s = jnp.pad(srcv.astype(jnp.int32).reshape(_TILES, _EPT),
                ((0, 0), (0, _EPTP - _EPT)))
    d = jnp.pad(dstv.astype(jnp.int32).reshape(_TILES, _EPT),
                ((0, 0), (0, _EPTP - _EPT)), constant_values=_DUMMY)
    return (s | (d << 16)).reshape(_TILES, _NB, _BLK)


def _ee_matrix(ee1, ee2):
    m = jnp.zeros((_CH, _EMBP), jnp.float32)
    m = m.at[0:6, :_EMB].set(ee1.astype(jnp.float32))
    m = m.at[8:11, :_EMB].set(ee2.astype(jnp.float32))
    return m


def kernel(x, edge_index, edge_attr, masked_atom_mask, params):
    x0 = x[:, 0].astype(jnp.int32).reshape(_N, 1)
    x1 = x[:, 1].astype(jnp.int32).reshape(_N, 1)
    pkp = _packed_slabs(edge_index[0], edge_index[1])
    cpkp = _packed_slabs(edge_attr[:, 0] * 3 + edge_attr[:, 1], edge_index[1])
    maskcol = masked_atom_mask.astype(jnp.int32).reshape(_N, 1)

    p = params
    e1 = _pad2(p['atom_emb1'], 128, _EMBP)
    e2 = _pad2(p['atom_emb2'], 8, _EMBP)
    e2d = _pad2(p['enc_to_dec'], _EMBP, _EMBP)
    alpha = p['prelu_alpha'].reshape(1, 1).astype(jnp.float32)

    t = np.zeros((24, _CH), np.float32)
    for k in range(9):
        t[k, k // 3] = 1.0
        t[k, 8 + k % 3] = 1.0
    cnt = _sc_cnt(jnp.asarray(t), cpkp)

    h = _tc_embed(x0, x1, e1, e2)
    for l, lp in enumerate(p['layers']):
        z = _sc_spmv(*h, pkp)
        ee = _ee_matrix(lp['ee1'], lp['ee2'])
        sl = jnp.pad((lp['ee1'][4] + lp['ee2'][0]).astype(jnp.float32),
                     (0, _EMBP - _EMB)).reshape(1, _EMBP)
        w1 = _pad2(lp['W1'], _EMBP, _HIDP)
        b1 = jnp.pad(lp['b1'].astype(jnp.float32),
                     (0, _HIDP - 2 * _EMB)).reshape(1, _HIDP)
        w2 = _pad2(lp['W2'], _HIDP, _EMBP)
        b2 = jnp.pad(lp['b2'].astype(jnp.float32),
                     (0, _EMBP - _EMB)).reshape(1, _EMBP)
        bns = jnp.pad(lp['bn_scale'].astype(jnp.float32),
                      (0, _EMBP - _EMB)).reshape(1, _EMBP)
        bnb = jnp.pad(lp['bn_bias'].astype(jnp.float32),
                      (0, _EMBP - _EMB)).reshape(1, _EMBP)
        h = _tc_dense(z, h, cnt, ee, sl, w1, b1, w2, b2, bns, bnb,
                      relu_out=(l != 4), nout=_NCH)

    d = _tc_decpre(h, maskcol, e2d, alpha)
    z = _sc_spmv(*d, pkp)
    dp = p['dec']
    ee = _ee_matrix(dp['ee1'], dp['ee2'])
    sl = jnp.pad((dp['ee1'][4] + dp['ee2'][0]).astype(jnp.float32),
                 (0, _EMBP - _EMB)).reshape(1, _EMBP)
    w1 = _pad2(dp['W1'], _EMBP, _HIDP)
    b1 = jnp.pad(dp['b1'].astype(jnp.float32),
                 (0, _HIDP - 2 * _EMB)).reshape(1, _HIDP)
    w2 = _pad2(dp['W2'], _HIDP, _CH)
    b2 = jnp.pad(dp['b2'].astype(jnp.float32), (0, _CH - _OUT)).reshape(1, _CH)
    out = _tc_dense(z, d, cnt, ee, sl, w1, b1, w2, b2, None, None,
                    relu_out=False, nout=1)
    return out[0][:, :_OUT]


# speedup vs baseline: 10.0159x; 1.2026x over previous
import functools

import numpy as np

import jax
import jax.numpy as jnp
from jax import lax
from jax.experimental import pallas as pl
from jax.experimental.pallas import tpu as pltpu
from jax.experimental.pallas import tpu_sc as plsc

_N = 10000
_E = 160000
_EMB = 500
_EMBP = 512
_HIDP = 1024
_OUT = 119
_CH = 128
_NCH = 4
_TILES = 16
_BLK = 128
_EPT = _E // _TILES
_NB = (_EPT + _BLK - 1) // _BLK
_EPTP = _NB * _BLK
_NPAD = 10112
_DUMMY = _N
_STRIPE = _NPAD // _TILES
_OSTRIPE = 624
_OTAIL = _N - _TILES * _OSTRIPE
_RB = 400
_GRID = _N // _RB

@functools.lru_cache(maxsize=None)
def _build_spmv(nch):
    mesh = plsc.VectorSubcoreMesh(core_axis_name="c", subcore_axis_name="s")
    return functools.partial(
        pl.kernel,
        out_type=[jax.ShapeDtypeStruct((_N, _CH), jnp.float32)] * nch,
        mesh=mesh,
        scratch_types=[
            pltpu.VMEM((_NB, _BLK), jnp.int32),
            pltpu.VMEM((4, _BLK), jnp.int32),
            pltpu.VMEM((_BLK, _CH), jnp.float32),
            pltpu.VMEM((_BLK, _CH), jnp.float32),
            pltpu.VMEM_SHARED((_NPAD, _CH), jnp.float32),
            pltpu.SemaphoreType.DMA,
            pltpu.SemaphoreType.DMA,
        ],
    )(functools.partial(_spmv_body, nch))


def _sc_spmv(h0, h1, h2, h3, pkp):
    return _build_spmv(_NCH)(h0, h1, h2, h3, pkp)


def _sc_cnt(tab, pkp):
    return _build_spmv(1)(tab, pkp)[0]


def _spmv_body(nch, *refs):
    hrefs = refs[0:nch]
    pkp = refs[nch]
    orefs = refs[nch + 1:2 * nch + 1]
    pk_v, ibuf, gb0, gb1, acc, sem0, sem1 = refs[2 * nch + 1:]
    cid = lax.axis_index("c")
    sid = lax.axis_index("s")
    pltpu.sync_copy(pkp.at[sid], pk_v)

    zero16 = jnp.zeros((16,), jnp.float32)
    m16 = jnp.full((16,), 0xFFFF, jnp.int32)

    def _unpack(j, slot):
        for g in range(_BLK // 16):
            v = pk_v[j, pl.ds(g * 16, 16)]
            ibuf[2 * slot, pl.ds(g * 16, 16)] = jnp.bitwise_and(v, m16)
            ibuf[2 * slot + 1, pl.ds(g * 16, 16)] = \
                lax.shift_right_logical(v, 16)

    for ch in range(nch):
        @pl.when(cid == ch // 2)
        def _(ch=ch):
            def _zrow(r, carry):
                for j in range(_CH // 16):
                    gb0[r, pl.ds(j * 16, 16)] = zero16
                return carry

            lax.fori_loop(0, _BLK, _zrow, 0)
            base = sid * _STRIPE
            nfull = _STRIPE // _BLK
            for z in range(nfull):
                pltpu.sync_copy(gb0, acc.at[pl.ds(base + z * _BLK, _BLK)])
            rem = _STRIPE - nfull * _BLK
            if rem:
                pltpu.sync_copy(gb0.at[pl.ds(0, rem)],
                                acc.at[pl.ds(base + nfull * _BLK, rem)])
            plsc.subcore_barrier()

            tab = hrefs[ch]

            def _wait(buf, sem):
                pltpu.make_async_copy(tab.at[ibuf.at[0]], buf, sem).wait()

            _unpack(0, 0)
            pltpu.async_copy(tab.at[ibuf.at[0]], gb0, sem0)

            def _pair(p, carry):
                j0 = 2 * p
                _unpack(j0 + 1, 1)
                pltpu.async_copy(tab.at[ibuf.at[2]], gb1, sem1)
                _wait(gb0, sem0)
                pltpu.sync_copy(gb0, acc.at[ibuf.at[1]], add=True)
                _unpack(j0 + 2, 0)
                pltpu.async_copy(tab.at[ibuf.at[0]], gb0, sem0)
                _wait(gb1, sem1)
                pltpu.sync_copy(gb1, acc.at[ibuf.at[3]], add=True)
                return carry

            lax.fori_loop(0, (_NB - 1) // 2, _pair, 0)
            _wait(gb0, sem0)
            pltpu.sync_copy(gb0, acc.at[ibuf.at[1]], add=True)
            plsc.subcore_barrier()
            pltpu.sync_copy(acc.at[pl.ds(sid * _OSTRIPE, _OSTRIPE)],
                            orefs[ch].at[pl.ds(sid * _OSTRIPE, _OSTRIPE)])

            @pl.when(sid == _TILES - 1)
            def _tail(ch=ch):
                pltpu.sync_copy(
                    acc.at[pl.ds(_TILES * _OSTRIPE, _OTAIL)],
                    orefs[ch].at[pl.ds(_TILES * _OSTRIPE, _OTAIL)])
            plsc.subcore_barrier()


def _full(shape):
    return pl.BlockSpec(shape, lambda i: (0,) * len(shape))


def _rows(shape):
    return pl.BlockSpec(shape, lambda i: (i,) + (0,) * (len(shape) - 1))


def _tc_embed(x0, x1, e1, e2):
    def body(x0r, x1r, e1r, e2r, o0, o1, o2, o3):
        oh1 = (x0r[...] == lax.broadcasted_iota(jnp.int32, (1, 128), 1))
        oh2 = (x1r[...] == lax.broadcasted_iota(jnp.int32, (1, 8), 1))
        h = jnp.dot(oh1.astype(jnp.float32), e1r[...],
                    preferred_element_type=jnp.float32)
        h = h + jnp.dot(oh2.astype(jnp.float32), e2r[...],
                        preferred_element_type=jnp.float32)
        for k, o in enumerate((o0, o1, o2, o3)):
            o[...] = h[:, k * _CH:(k + 1) * _CH]

    return pl.pallas_call(
        body,
        grid=(_GRID,),
        in_specs=[_rows((_RB, 1)), _rows((_RB, 1)),
                  _full((128, _EMBP)), _full((8, _EMBP))],
        out_specs=[_rows((_RB, _CH))] * _NCH,
        out_shape=[jax.ShapeDtypeStruct((_N, _CH), jnp.float32)] * _NCH,
    )(x0, x1, e1, e2)


def _tc_dense(z, h, cnt, ee, sl, w1, b1, w2, b2, bns, bnb, relu_out, nout):
    def body(*refs):
        zr = refs[0:4]
        hr = refs[4:8]
        cr, eer, slr, w1r, b1r, w2r, b2r = refs[8:15]
        pos = 15
        if bns is not None:
            bnsr, bnbr = refs[15:17]
            pos = 17
        outs = refs[pos:]
        agg = jnp.concatenate([a[...] + b[...] for a, b in zip(zr, hr)],
                              axis=1)
        agg = agg + slr[...] + jnp.dot(cr[...], eer[...],
                                       preferred_element_type=jnp.float32)
        hid = jnp.maximum(jnp.dot(agg, w1r[...],
                                  preferred_element_type=jnp.float32)
                          + b1r[...], 0.0)
        o = jnp.dot(hid, w2r[...], preferred_element_type=jnp.float32) \
            + b2r[...]
        if bns is not None:
            o = o * bnsr[...] + bnbr[...]
        if relu_out:
            o = jnp.maximum(o, 0.0)
        for k, oref in enumerate(outs):
            oref[...] = o[:, k * _CH:(k + 1) * _CH]

    ncols = nout * _CH
    in_specs = ([_rows((_RB, _CH))] * 8
                + [_rows((_RB, _CH)), _full((_CH, _EMBP)), _full((1, _EMBP)),
                   _full((_EMBP, _HIDP)), _full((1, _HIDP)),
                   _full((_HIDP, ncols)), _full((1, ncols))])
    args = list(z) + list(h) + [cnt, ee, sl, w1, b1, w2, b2]
    if bns is not None:
        in_specs += [_full((1, ncols)), _full((1, ncols))]
        args += [bns, bnb]
    return pl.pallas_call(
        body,
        grid=(_GRID,),
        in_specs=in_specs,
        out_specs=[_rows((_RB, _CH))] * nout,
        out_shape=[jax.ShapeDtypeStruct((_N, _CH), jnp.float32)] * nout,
    )(*args)


def _tc_decpre(h, mask, e2d, alpha):
    def body(h0r, h1r, h2r, h3r, mr, er, ar, o0, o1, o2, o3):
        hcat = jnp.concatenate([r[...] for r in (h0r, h1r, h2r, h3r)], axis=1)
        a = ar[0, 0]
        p = jnp.where(hcat >= 0.0, hcat, a * hcat)
        d = jnp.dot(p, er[...], preferred_element_type=jnp.float32)
        d = jnp.where(mr[...] > 0, 0.0, d)
        for k, o in enumerate((o0, o1, o2, o3)):
            o[...] = d[:, k * _CH:(k + 1) * _CH]

    return pl.pallas_call(
        body,
        grid=(_GRID,),
        in_specs=[_rows((_RB, _CH))] * 4
        + [_rows((_RB, 1)), _full((_EMBP, _EMBP)), _full((1, 1))],
        out_specs=[_rows((_RB, _CH))] * _NCH,
        out_shape=[jax.ShapeDtypeStruct((_N, _CH), jnp.float32)] * _NCH,
    )(*h, mask, e2d, alpha)


def _pad2(a, r, c):
    return jnp.pad(a.astype(jnp.float32), ((0, r - a.shape[0]),
                                           (0, c - a.shape[1])))


def _packed_slabs(srcv, dstv):
    s = jnp.pad(srcv.astype(jnp.int32).reshape(_TILES, _EPT),
                ((0, 0), (0, _EPTP - _EPT)))
    d = jnp.pad(dstv.astype(jnp.int32).reshape(_TILES, _EPT),
                ((0, 0), (0, _EPTP - _EPT)), constant_values=_DUMMY)
    return (s | (d << 16)).reshape(_TILES, _NB, _BLK)


def _ee_matrix(ee1, ee2):
    m = jnp.zeros((_CH, _EMBP), jnp.float32)
    m = m.at[0:6, :_EMB].set(ee1.astype(jnp.float32))
    m = m.at[8:11, :_EMB].set(ee2.astype(jnp.float32))
    return m


def kernel(x, edge_index, edge_attr, masked_atom_mask, params):
    x0 = x[:, 0].astype(jnp.int32).reshape(_N, 1)
    x1 = x[:, 1].astype(jnp.int32).reshape(_N, 1)
    pkp = _packed_slabs(edge_index[0], edge_index[1])
    combo = (edge_attr[:, 0] * 3 + edge_attr[:, 1]).astype(jnp.int32)
    combo = (combo.reshape(_TILES, _EPT)
             + 24 * jnp.arange(_TILES, dtype=jnp.int32)[:, None]).reshape(-1)
    cpkp = _packed_slabs(combo, edge_index[1])
    maskcol = masked_atom_mask.astype(jnp.int32).reshape(_N, 1)

    p = params
    e1 = _pad2(p['atom_emb1'], 128, _EMBP)
    e2 = _pad2(p['atom_emb2'], 8, _EMBP)
    e2d = _pad2(p['enc_to_dec'], _EMBP, _EMBP)
    alpha = p['prelu_alpha'].reshape(1, 1).astype(jnp.float32)

    t = np.zeros((24, _CH), np.float32)
    for k in range(9):
        t[k, k // 3] = 1.0
        t[k, 8 + k % 3] = 1.0
    cnt = _sc_cnt(jnp.asarray(np.tile(t, (_TILES, 1))), cpkp)

    h = _tc_embed(x0, x1, e1, e2)
    for l, lp in enumerate(p['layers']):
        z = _sc_spmv(*h, pkp)
        ee = _ee_matrix(lp['ee1'], lp['ee2'])
        sl = jnp.pad((lp['ee1'][4] + lp['ee2'][0]).astype(jnp.float32),
                     (0, _EMBP - _EMB)).reshape(1, _EMBP)
        w1 = _pad2(lp['W1'], _EMBP, _HIDP)
        b1 = jnp.pad(lp['b1'].astype(jnp.float32),
                     (0, _HIDP - 2 * _EMB)).reshape(1, _HIDP)
        w2 = _pad2(lp['W2'], _HIDP, _EMBP)
        b2 = jnp.pad(lp['b2'].astype(jnp.float32),
                     (0, _EMBP - _EMB)).reshape(1, _EMBP)
        bns = jnp.pad(lp['bn_scale'].astype(jnp.float32),
                      (0, _EMBP - _EMB)).reshape(1, _EMBP)
        bnb = jnp.pad(lp['bn_bias'].astype(jnp.float32),
                      (0, _EMBP - _EMB)).reshape(1, _EMBP)
        h = _tc_dense(z, h, cnt, ee, sl, w1, b1, w2, b2, bns, bnb,
                      relu_out=(l != 4), nout=_NCH)

    d = _tc_decpre(h, maskcol, e2d, alpha)
    z = _sc_spmv(*d, pkp)
    dp = p['dec']
    ee = _ee_matrix(dp['ee1'], dp['ee2'])
    sl = jnp.pad((dp['ee1'][4] + dp['ee2'][0]).astype(jnp.float32),
                 (0, _EMBP - _EMB)).reshape(1, _EMBP)
    w1 = _pad2(dp['W1'], _EMBP, _HIDP)
    b1 = jnp.pad(dp['b1'].astype(jnp.float32),
                 (0, _HIDP - 2 * _EMB)).reshape(1, _HIDP)
    w2 = _pad2(dp['W2'], _HIDP, _CH)
    b2 = jnp.pad(dp['b2'].astype(jnp.float32), (0, _CH - _OUT)).reshape(1, _CH)
    out = _tc_dense(z, d, cnt, ee, sl, w1, b1, w2, b2, None, None,
                    relu_out=False, nout=1)
    return out[0][:, :_OUT]
